# baseline (speedup 1.0000x reference)
import jax
import jax.numpy as jnp
from jax.experimental import pallas as pl


BLOCK_N = 256


def _add_kernel(x_ref, t_ref, o_ref):
    o_ref[...] = x_ref[...] + t_ref[...]


def kernel(inputs, table):
    b, f, d = inputs.shape
    x2 = jnp.transpose(inputs, (1, 2, 0)).reshape(f * d, b)
    t2 = table.reshape(f * d, 1)

    out2 = pl.pallas_call(
        _add_kernel,
        grid=(b // BLOCK_N,),
        in_specs=[
            pl.BlockSpec((f * d, BLOCK_N), lambda i: (0, i)),
            pl.BlockSpec((f * d, 1), lambda i: (0, 0)),
        ],
        out_specs=pl.BlockSpec((f * d, BLOCK_N), lambda i: (0, i)),
        out_shape=jax.ShapeDtypeStruct((f * d, b), inputs.dtype),
    )(x2, t2)
    return jnp.transpose(out2.reshape(f, d, b), (2, 0, 1))


# speedup vs baseline: 1.0587x; 1.0587x over previous
import jax
import jax.numpy as jnp
from jax.experimental import pallas as pl


BLOCK_R = 200


def _add_kernel(x_ref, t_ref, o_ref):
    o_ref[...] = x_ref[...] + t_ref[...]


def kernel(inputs, table):
    b, f, d = inputs.shape
    x2 = jnp.transpose(inputs, (1, 2, 0)).reshape(f * d, b)
    t2 = table.reshape(f * d, 1)

    out2 = pl.pallas_call(
        _add_kernel,
        grid=(f * d // BLOCK_R,),
        in_specs=[
            pl.BlockSpec((BLOCK_R, b), lambda i: (i, 0)),
            pl.BlockSpec((BLOCK_R, 1), lambda i: (i, 0)),
        ],
        out_specs=pl.BlockSpec((BLOCK_R, b), lambda i: (i, 0)),
        out_shape=jax.ShapeDtypeStruct((f * d, b), inputs.dtype),
    )(x2, t2)
    return jnp.transpose(out2.reshape(f, d, b), (2, 0, 1))


# speedup vs baseline: 1.0587x; 1.0000x over previous
import jax
import jax.numpy as jnp
from jax.experimental import pallas as pl


BLOCK_R = 160


def _add_kernel(x_ref, t_ref, o_ref):
    o_ref[...] = x_ref[...] + t_ref[...]


def kernel(inputs, table):
    b, f, d = inputs.shape
    x2 = jnp.transpose(inputs, (1, 2, 0)).reshape(f * d, b)
    t2 = table.reshape(f * d, 1)

    out2 = pl.pallas_call(
        _add_kernel,
        grid=(f * d // BLOCK_R,),
        in_specs=[
            pl.BlockSpec((BLOCK_R, b), lambda i: (i, 0)),
            pl.BlockSpec((BLOCK_R, 1), lambda i: (i, 0)),
        ],
        out_specs=pl.BlockSpec((BLOCK_R, b), lambda i: (i, 0)),
        out_shape=jax.ShapeDtypeStruct((f * d, b), inputs.dtype),
    )(x2, t2)
    return jnp.transpose(out2.reshape(f, d, b), (2, 0, 1))
